# Initial kernel scaffold; baseline (speedup 1.0000x reference)
import jax
import jax.numpy as jnp
from jax.experimental import pallas as pl


def kernel(inputs, embedding):
    raise NotImplementedError("write your pallas kernel here")


# speedup vs baseline: 1.6845x; 1.6845x over previous
import functools

import jax
import jax.numpy as jnp
from jax import lax
from jax.experimental import pallas as pl
from jax.experimental.pallas import tpu as pltpu
from jax.experimental.pallas import tpu_sc as plsc

_NC = 2
_NS = 16
_NW = _NC * _NS

_BATCH = 16384
_HIST = 50
_FEATURES = 64
_TOTAL = _BATCH * _HIST
_PER_W = _TOTAL // _NW
_G = 128
_NG = _PER_W // _G


def _embed_gather(idx3, table):
  mesh = plsc.VectorSubcoreMesh(core_axis_name="c", subcore_axis_name="s")

  @functools.partial(
      pl.kernel,
      mesh=mesh,
      compiler_params=pltpu.CompilerParams(use_tc_tiling_on_sc=False),
      out_type=jax.ShapeDtypeStruct((_TOTAL, _FEATURES), jnp.float32),
      scratch_types=[
          pltpu.VMEM((_NG, _G), jnp.int32),
          pltpu.VMEM((_G, _FEATURES), jnp.float32),
          pltpu.SemaphoreType.DMA,
      ],
  )
  def k(idx_hbm, table_hbm, out_hbm, idx_v, rows_v, sem):
    wid = lax.axis_index("s") * _NC + lax.axis_index("c")
    base = wid * _PER_W
    pltpu.sync_copy(idx_hbm.at[wid], idx_v)

    def body(j, carry):
      pltpu.async_copy(table_hbm.at[idx_v.at[j]], rows_v, sem).wait()
      pltpu.sync_copy(rows_v, out_hbm.at[pl.ds(base + j * _G, _G)])
      return carry

    lax.fori_loop(0, _NG, body, 0)

  return k(idx3, table)


def kernel(inputs, embedding):
  idx3 = inputs.reshape(_NW, _NG, _G).astype(jnp.int32)
  out = _embed_gather(idx3, embedding)
  return out.reshape(_BATCH, _HIST, _FEATURES)


# speedup vs baseline: 1.8774x; 1.1145x over previous
import functools

import jax
import jax.numpy as jnp
from jax import lax
from jax.experimental import pallas as pl
from jax.experimental.pallas import tpu as pltpu
from jax.experimental.pallas import tpu_sc as plsc

_NC = 2
_NS = 16
_NW = _NC * _NS

_BATCH = 16384
_HIST = 50
_FEATURES = 64
_TOTAL = _BATCH * _HIST
_PER_W = _TOTAL // _NW
_G = 128
_NG = _PER_W // _G
_K = 5
_M_ROWS = _K * _G
_MACROS = _NG // _K


def _embed_gather(idx3, table):
  mesh = plsc.VectorSubcoreMesh(core_axis_name="c", subcore_axis_name="s")

  @functools.partial(
      pl.kernel,
      mesh=mesh,
      compiler_params=pltpu.CompilerParams(use_tc_tiling_on_sc=False),
      out_type=jax.ShapeDtypeStruct((_TOTAL, _FEATURES), jnp.float32),
      scratch_types=[
          pltpu.VMEM((_NG, _G), jnp.int32),
          pltpu.VMEM((_M_ROWS, _FEATURES), jnp.float32),
          pltpu.VMEM((_M_ROWS, _FEATURES), jnp.float32),
          pltpu.SemaphoreType.DMA,
          pltpu.SemaphoreType.DMA,
          pltpu.SemaphoreType.DMA,
          pltpu.SemaphoreType.DMA,
      ],
  )
  def k(idx_hbm, table_hbm, out_hbm, idx_v, rows0, rows1, sg0, sg1, sw0, sw1):
    wid = lax.axis_index("s") * _NC + lax.axis_index("c")
    base = wid * _PER_W
    rows = (rows0, rows1)
    sg = (sg0, sg1)
    sw = (sw0, sw1)

    pltpu.sync_copy(idx_hbm.at[wid], idx_v)

    def fire_gathers(m, b):
      for kk in range(_K):
        pltpu.async_copy(
            table_hbm.at[idx_v.at[_K * m + kk]],
            rows[b].at[pl.ds(kk * _G, _G)],
            sg[b])

    def drain_gathers(b):
      pltpu.make_async_copy(out_hbm.at[pl.ds(0, _M_ROWS)], rows[b], sg[b]).wait()

    def fire_write(m, b):
      pltpu.async_copy(rows[b], out_hbm.at[pl.ds(base + m * _M_ROWS, _M_ROWS)], sw[b])

    def drain_write(b):
      pltpu.make_async_copy(rows[b], out_hbm.at[pl.ds(base, _M_ROWS)], sw[b]).wait()

    fire_gathers(0, 0)
    fire_gathers(1, 1)
    drain_gathers(0)
    fire_write(0, 0)

    def body(m2, carry):
      for h in range(2):
        m = 2 * m2 + h
        drain_write(h)
        fire_gathers(m, h)
        drain_gathers(1 - h)
        fire_write(m - 1, 1 - h)
      return carry

    lax.fori_loop(1, _MACROS // 2, body, 0)

    drain_gathers(1)
    fire_write(_MACROS - 1, 1)
    drain_write(0)
    drain_write(1)

  return k(idx3, table)


def kernel(inputs, embedding):
  idx3 = inputs.reshape(_NW, _NG, _G).astype(jnp.int32)
  out = _embed_gather(idx3, embedding)
  return out.reshape(_BATCH, _HIST, _FEATURES)
